# baseline (speedup 1.0000x reference)
import functools
import jax
import jax.numpy as jnp
from jax import lax
from jax.experimental import pallas as pl
from jax.experimental.pallas import tpu as pltpu
from jax.experimental.pallas import tpu_sc as plsc

NC, NS, L = 2, 16, 16
NW = NC * NS
N_TOTAL = 16384 * 200
PER_W = N_TOTAL // NW
CHUNK = 12800
NCHUNK = PER_W // CHUNK
UNROLL = 4
VEC_ITERS = CHUNK // (16 * UNROLL)


def _rowsum_body(emb_ref, out_ref):
    out_ref[...] = jnp.sum(emb_ref[...], axis=0, keepdims=True)


def _finish_body(parts_ref, out_ref):
    out_ref[...] = jnp.sum(parts_ref[...], keepdims=True).reshape(1, 1) - 1.0


def _sc_body(idx_hbm, rs_hbm, out_hbm, rs_v, bufa, bufb, acc_v, sem_rs, sema, semb):
    wid = lax.axis_index("s") * NC + lax.axis_index("c")
    base = wid * PER_W

    pltpu.async_copy(rs_hbm, rs_v, sem_rs).wait()

    bufs = (bufa, bufb)
    sems = (sema, semb)
    copies = [None, None]
    copies[0] = pltpu.async_copy(
        idx_hbm.at[pl.ds(base, CHUNK)], bufa, sema)

    zero = jnp.zeros((16,), jnp.float32)
    accs = (zero, zero, zero, zero)

    for k in range(NCHUNK):
        cur = k % 2
        if k + 1 < NCHUNK:
            copies[1 - cur] = pltpu.async_copy(
                idx_hbm.at[pl.ds(base + (k + 1) * CHUNK, CHUNK)],
                bufs[1 - cur], sems[1 - cur])
        copies[cur].wait()
        buf = bufs[cur]

        def body(j, accs):
            a0, a1, a2, a3 = accs
            off = j * (16 * UNROLL)
            g0 = plsc.load_gather(rs_v, [buf[pl.ds(off, 16)]])
            g1 = plsc.load_gather(rs_v, [buf[pl.ds(off + 16, 16)]])
            g2 = plsc.load_gather(rs_v, [buf[pl.ds(off + 32, 16)]])
            g3 = plsc.load_gather(rs_v, [buf[pl.ds(off + 48, 16)]])
            return (a0 + g0, a1 + g1, a2 + g2, a3 + g3)

        accs = lax.fori_loop(0, VEC_ITERS, body, accs)

    acc_v[...] = accs[0] + accs[1] + accs[2] + accs[3]
    pltpu.sync_copy(acc_v, out_hbm.at[wid])


_sc_partials = functools.partial(
    pl.kernel,
    mesh=plsc.VectorSubcoreMesh(core_axis_name="c", subcore_axis_name="s"),
    out_type=jax.ShapeDtypeStruct((NW, 16), jnp.float32),
    scratch_types=[
        pltpu.VMEM((128,), jnp.float32),
        pltpu.VMEM((CHUNK,), jnp.int32),
        pltpu.VMEM((CHUNK,), jnp.int32),
        pltpu.VMEM((16,), jnp.float32),
        pltpu.SemaphoreType.DMA,
        pltpu.SemaphoreType.DMA,
        pltpu.SemaphoreType.DMA,
    ],
    compiler_params=pltpu.CompilerParams(needs_layout_passes=False),
)(_sc_body)


def kernel(input_batch, emb36a, emb36b, emb48):
    del emb36a, emb36b
    flat = input_batch.reshape(-1).astype(jnp.int32)
    emb_t = jnp.zeros((emb48.shape[1], 128), jnp.float32).at[:, : emb48.shape[0]].set(emb48.T)
    rs = pl.pallas_call(
        _rowsum_body,
        out_shape=jax.ShapeDtypeStruct((1, 128), jnp.float32),
    )(emb_t)
    parts = _sc_partials(flat, rs.reshape(128))
    out = pl.pallas_call(
        _finish_body,
        out_shape=jax.ShapeDtypeStruct((1, 1), jnp.float32),
    )(parts)
    return out[0, 0]


# speedup vs baseline: 2.2414x; 2.2414x over previous
import jax
import jax.numpy as jnp
from jax.experimental import pallas as pl


_GRID = 4


def _body(idx_ref, emb_ref, out_ref):
    i = pl.program_id(0)
    rs = jax.lax.dot_general(
        jnp.ones((1, emb_ref.shape[1]), jnp.float32),
        emb_ref[...],
        (((1,), (1,)), ((), ())),
        preferred_element_type=jnp.float32,
    )
    idx = idx_ref[...]
    table = jnp.broadcast_to(rs, (idx.shape[0], rs.shape[1]))
    vals = jnp.take_along_axis(table, idx, axis=1)
    part = jnp.sum(vals, keepdims=True).reshape(1, 1)

    @pl.when(i == 0)
    def _():
        out_ref[...] = part - 1.0

    @pl.when(i > 0)
    def _():
        out_ref[...] += part


def kernel(input_batch, emb36a, emb36b, emb48):
    del emb36a, emb36b
    n, c = input_batch.shape
    block = n // _GRID
    out = pl.pallas_call(
        _body,
        grid=(_GRID,),
        in_specs=[
            pl.BlockSpec((block, c), lambda i: (i, 0)),
            pl.BlockSpec(emb48.shape, lambda i: (0, 0)),
        ],
        out_specs=pl.BlockSpec((1, 1), lambda i: (0, 0)),
        out_shape=jax.ShapeDtypeStruct((1, 1), jnp.float32),
    )(input_batch, emb48)
    return out.reshape(())
